# baseline (speedup 1.0000x reference)
import functools

import jax
import jax.numpy as jnp
from jax import lax
from jax.experimental import pallas as pl
from jax.experimental.pallas import tpu as pltpu
from jax.experimental.pallas import tpu_sc as plsc

VOCAB = 1000000
EMBED_DIM = 64
BATCH = 16384
NUM_CLASSES = 128

NW = 32
PER_W = BATCH // NW
GROUP = 16
NGROUP = PER_W // GROUP
BM = 2048
NB = BATCH // BM


def _sc_gather(table, idx):
    mesh = plsc.VectorSubcoreMesh(core_axis_name="c", subcore_axis_name="s")

    @functools.partial(
        pl.kernel,
        out_type=jax.ShapeDtypeStruct((BATCH, EMBED_DIM), jnp.float32),
        mesh=mesh,
        scratch_types=[
            pltpu.VMEM((PER_W,), jnp.int32),
            pltpu.SemaphoreType.DMA,
        ],
    )
    def k(table_hbm, idx_hbm, out_hbm, idx_v, sem):
        wid = lax.axis_index("s") * 2 + lax.axis_index("c")
        base = wid * PER_W
        pltpu.sync_copy(idx_hbm.at[pl.ds(base, PER_W)], idx_v)

        def group_body(g, _):
            ivec = idx_v[pl.ds(g * GROUP, GROUP)]
            copies = []
            for kk in range(GROUP):
                j = g * GROUP + kk
                r = ivec[kk]
                copies.append(pltpu.make_async_copy(
                    table_hbm.at[pl.ds(r, 1), :],
                    out_hbm.at[pl.ds(base + j, 1), :],
                    sem,
                ))
            for c in copies:
                c.start()
            for c in copies:
                c.wait()
            return 0

        lax.fori_loop(0, NGROUP, group_body, 0)

    return k(table, idx)


def _tc_body(x_ref, w_ref, b_ref, lab_ref, out_ref):
    i = pl.program_id(0)
    x = x_ref[...]
    w = w_ref[...]
    bias = b_ref[...]
    lab = lab_ref[0, 0, :]
    scores = jnp.dot(x, w, preferred_element_type=jnp.float32) + bias
    m = jnp.max(scores, axis=-1, keepdims=True)
    lse = jnp.log(jnp.sum(jnp.exp(scores - m), axis=-1, keepdims=True)) + m
    cls = lax.broadcasted_iota(jnp.int32, scores.shape, 1)
    picked = jnp.sum(
        jnp.where(cls == lab[:, None], scores, 0.0), axis=-1, keepdims=True
    )
    part = jnp.sum(lse - picked, axis=0, keepdims=True) * (1.0 / BATCH)

    @pl.when(i == 0)
    def _():
        out_ref[...] = part

    @pl.when(i > 0)
    def _():
        out_ref[...] = out_ref[...] + part


def _tc_loss(x, w, bias, labels3):
    return pl.pallas_call(
        _tc_body,
        grid=(NB,),
        in_specs=[
            pl.BlockSpec((BM, EMBED_DIM), lambda i: (i, 0)),
            pl.BlockSpec((EMBED_DIM, NUM_CLASSES), lambda i: (0, 0)),
            pl.BlockSpec((1, NUM_CLASSES), lambda i: (0, 0)),
            pl.BlockSpec((1, 1, BM), lambda i: (i, 0, 0)),
        ],
        out_specs=pl.BlockSpec((1, 1), lambda i: (0, 0)),
        out_shape=jax.ShapeDtypeStruct((1, 1), jnp.float32),
    )(x, w, bias, labels3)


def kernel(sentence_features, labels, emb, W, b):
    idx = sentence_features.astype(jnp.int32)
    x = _sc_gather(emb * jnp.float32(1.0000001), idx)
    labels3 = labels.astype(jnp.int32).reshape(NB, 1, BM)
    loss = _tc_loss(x, W, b.reshape(1, NUM_CLASSES), labels3)
    return loss[0, 0]


# speedup vs baseline: 2.0267x; 2.0267x over previous
import jax
import jax.numpy as jnp
from jax import lax
from jax.experimental import pallas as pl
from jax.experimental.pallas import tpu as pltpu

VOCAB = 1000000
EMBED_DIM = 64
BATCH = 16384
NUM_CLASSES = 128

BM = 2048
NB = BATCH // BM
NQ = 2
QTR = BM // NQ
UNROLL = 4


def _body(idx_sref, emb_ref, w_ref, b_ref, lab_ref, out_ref, xbuf, sems):
    i = pl.program_id(0)

    def issue_block(block, slot):
        def issue_one(j, _):
            for q in range(NQ):
                r = idx_sref[block * BM + q * QTR + j]
                pltpu.make_async_copy(
                    emb_ref.at[pl.ds(r, 1), :],
                    xbuf.at[slot, pl.ds(q * QTR + j, 1), :],
                    sems.at[slot, q],
                ).start(priority=q)
            return 0
        lax.fori_loop(0, QTR, issue_one, 0, unroll=UNROLL)

    @pl.when(i == 0)
    def _():
        issue_block(0, 0)

    @pl.when(i + 1 < NB)
    def _():
        issue_block(i + 1, (i + 1) % 2)

    slot = i % 2
    for q in range(NQ):
        pltpu.make_async_copy(
            emb_ref.at[pl.ds(0, QTR), :],
            xbuf.at[slot, pl.ds(q * QTR, QTR), :],
            sems.at[slot, q],
        ).wait()

    x = xbuf[slot]
    w = w_ref[...]
    bias = b_ref[...]
    lab = lab_ref[0, 0, :]
    scores = jnp.dot(x, w, preferred_element_type=jnp.float32) + bias
    m = jnp.max(scores, axis=-1, keepdims=True)
    lse = jnp.log(jnp.sum(jnp.exp(scores - m), axis=-1, keepdims=True)) + m
    cls = lax.broadcasted_iota(jnp.int32, scores.shape, 1)
    picked = jnp.sum(
        jnp.where(cls == lab[:, None], scores, 0.0), axis=-1, keepdims=True
    )
    part = jnp.sum(lse - picked, axis=0, keepdims=True) * (1.0 / BATCH)

    @pl.when(i == 0)
    def _():
        out_ref[...] = part

    @pl.when(i > 0)
    def _():
        out_ref[...] = out_ref[...] + part


def kernel(sentence_features, labels, emb, W, b):
    idx = sentence_features.astype(jnp.int32)
    labels3 = labels.astype(jnp.int32).reshape(NB, 1, BM)
    grid_spec = pltpu.PrefetchScalarGridSpec(
        num_scalar_prefetch=1,
        grid=(NB,),
        in_specs=[
            pl.BlockSpec(memory_space=pltpu.HBM),
            pl.BlockSpec((EMBED_DIM, NUM_CLASSES), lambda i, *_: (0, 0)),
            pl.BlockSpec((1, NUM_CLASSES), lambda i, *_: (0, 0)),
            pl.BlockSpec((1, 1, BM), lambda i, *_: (i, 0, 0)),
        ],
        out_specs=pl.BlockSpec((1, 1), lambda i, *_: (0, 0)),
        scratch_shapes=[
            pltpu.VMEM((2, BM, EMBED_DIM), jnp.float32),
            pltpu.SemaphoreType.DMA((2, NQ)),
        ],
    )
    loss = pl.pallas_call(
        _body,
        grid_spec=grid_spec,
        out_shape=jax.ShapeDtypeStruct((1, 1), jnp.float32),
    )(idx, emb, W, b.reshape(1, NUM_CLASSES), labels3)
    return loss[0, 0]
